# Initial kernel scaffold; baseline (speedup 1.0000x reference)
import jax
import jax.numpy as jnp
from jax.experimental import pallas as pl


def kernel(x, params):
    raise NotImplementedError("write your pallas kernel here")


# speedup vs baseline: 45.0628x; 45.0628x over previous
import jax
import jax.numpy as jnp
from jax.experimental import pallas as pl
from jax.experimental.pallas import tpu as pltpu

_B, _N, _D = 2, 1024, 768
_K = 512
_C = 151
_QB = 256
_EPS = 1e-5
_F32 = jnp.float32


def _gelu(x):
    return 0.5 * x * (1.0 + jax.lax.erf(x * (2.0 ** -0.5)))


def _ln(x, g=None, b=None):
    m = jnp.mean(x, axis=-1, keepdims=True)
    xc = x - m
    v = jnp.mean(xc * xc, axis=-1, keepdims=True)
    y = xc * jax.lax.rsqrt(v + _EPS)
    if g is not None:
        y = y * g + b
    return y


def _mm_t(a, w):
    return jax.lax.dot_general(a, w, (((1,), (1,)), ((), ())),
                               preferred_element_type=_F32)


def _qk_kernel(x_ref,
               sw1, sb1, sw2, sb2, sw3, sb3, sg, sbe,
               ow1, ob1, ow2, ob2, ow3, ob3, og, obe,
               q_ref, k_ref):
    x = x_ref[0]

    def head(w1, b1, w2, b2, w3, b3, g, be):
        h = _gelu(_mm_t(x, w1[...]) + b1[...])
        h = _gelu(_mm_t(h, w2[...]) + b2[...])
        h = _gelu(_mm_t(h, w3[...]) + b3[...])
        return _ln(h, g[...], be[...])

    q_ref[0] = x + head(sw1, sb1, sw2, sb2, sw3, sb3, sg, sbe)
    k_ref[0] = x + head(ow1, ob1, ow2, ob2, ow3, ob3, og, obe)


def _sel_kernel(q_ref, k_ref, w1, b1, w2, b2, g, be, cw, cb, bw, bb,
                probs_ref, bbox_ref):
    q = q_ref[0]
    k = k_ref[0]

    st = jax.lax.dot_general(k, q, (((1,), (1,)), ((), ())),
                             preferred_element_type=_F32)
    ii = jax.lax.broadcasted_iota(jnp.int32, (_N, _N), 0)
    jj = jax.lax.broadcasted_iota(jnp.int32, (_N, _N), 1)
    m = jnp.max(st, axis=0, keepdims=True)
    z = jnp.sum(jnp.exp(st - m), axis=0, keepdims=True)
    sd = jnp.sum(jnp.where(ii == jj, st, 0.0), axis=0, keepdims=True)
    d = jnp.exp(sd - m) / z

    diag_d = jnp.where(ii == jj, d, 0.0)
    dcol = jnp.dot(diag_d, jnp.ones((_N, 128), _F32),
                   preferred_element_type=_F32)
    di = pltpu.repeat(dcol, 8, axis=1)
    beats1 = jnp.where((d > di) | ((d == di) & (jj < ii)), 1.0, 0.0)
    rank_col = jnp.dot(beats1, jnp.ones((_N, 128), _F32),
                       preferred_element_type=_F32)
    sel_col = jnp.where(rank_col < float(_K), 1.0, 0.0)
    sel_coln = pltpu.repeat(sel_col, 8, axis=1)
    beats2 = jnp.where(ii == jj, 0.0, 1.0 - beats1)
    rank_row = jnp.sum(beats2, axis=0, keepdims=True)
    sel_row = jnp.where(rank_row < float(_K), 1.0, 0.0)
    psel = jnp.sum(jnp.where(ii <= jj, sel_coln, 0.0),
                   axis=0, keepdims=True)

    ss = jax.lax.broadcasted_iota(jnp.int32, (_K, _N), 0).astype(_F32)
    oh = jnp.where((sel_row > 0.0) & (psel == ss + 1.0), 1.0, 0.0)
    q_sel = jnp.dot(oh, q, preferred_element_type=_F32)

    h = _ln(q_sel + q_sel)
    h = _gelu(_mm_t(h, w1[...]) + b1[...])
    h = _mm_t(h, w2[...]) + b2[...]
    o = _ln(h, g[...], be[...])

    bbox_ref[0] = jax.nn.relu(_mm_t(o, bw[...]) + bb[...])
    lg = _mm_t(o, cw[...]) + cb[...]
    lm = jnp.max(lg, axis=-1, keepdims=True)
    e = jnp.exp(lg - lm)
    probs_ref[0] = e / jnp.sum(e, axis=-1, keepdims=True)


def _full_spec(shape):
    return pl.BlockSpec(shape, lambda *_: (0,) * len(shape))


def kernel(x, params):
    sh, oh_, m2 = params['subject_head'], params['object_head'], params['mlp2']
    r = lambda v: v.reshape(1, -1)

    w_args = []
    specs_w = []
    for p in (sh, oh_):
        for i in (1, 2, 3):
            w_args += [p[f'w{i}'], r(p[f'b{i}'])]
            specs_w += [_full_spec((_D, _D)), _full_spec((1, _D))]
        w_args += [r(p['g']), r(p['be'])]
        specs_w += [_full_spec((1, _D)), _full_spec((1, _D))]

    q, k = pl.pallas_call(
        _qk_kernel,
        grid=(_B, _N // _QB),
        in_specs=[pl.BlockSpec((1, _QB, _D), lambda b, rr: (b, rr, 0))] + specs_w,
        out_specs=[pl.BlockSpec((1, _QB, _D), lambda b, rr: (b, rr, 0))] * 2,
        out_shape=[jax.ShapeDtypeStruct((_B, _N, _D), _F32)] * 2,
        compiler_params=pltpu.CompilerParams(
            dimension_semantics=("parallel", "arbitrary"),
            vmem_limit_bytes=50 * 1024 * 1024,
        ),
    )(x, *w_args)

    probs, bbox = pl.pallas_call(
        _sel_kernel,
        grid=(_B,),
        in_specs=[pl.BlockSpec((1, _N, _D), lambda b: (b, 0, 0))] * 2 + [
            _full_spec((_D, _D)), _full_spec((1, _D)),
            _full_spec((_D, _D)), _full_spec((1, _D)),
            _full_spec((1, _D)), _full_spec((1, _D)),
            _full_spec((_C, _D)), _full_spec((1, _C)),
            _full_spec((4, _D)), _full_spec((1, 4)),
        ],
        out_specs=[pl.BlockSpec((1, _K, _C), lambda b: (b, 0, 0)),
                   pl.BlockSpec((1, _K, 4), lambda b: (b, 0, 0))],
        out_shape=[jax.ShapeDtypeStruct((_B, _K, _C), _F32),
                   jax.ShapeDtypeStruct((_B, _K, 4), _F32)],
        compiler_params=pltpu.CompilerParams(
            dimension_semantics=("parallel",),
            vmem_limit_bytes=60 * 1024 * 1024,
        ),
    )(q, k, m2['w1'], r(m2['b1']), m2['w2'], r(m2['b2']), r(m2['g']), r(m2['be']),
      params['cls_w'], r(params['cls_b']), params['bbox_w'], r(params['bbox_b']))

    return probs, bbox


# speedup vs baseline: 47.6413x; 1.0572x over previous
import jax
import jax.numpy as jnp
from jax.experimental import pallas as pl
from jax.experimental.pallas import tpu as pltpu

_B, _N, _D = 2, 1024, 768
_K = 512
_C = 151
_EPS = 1e-5
_F32 = jnp.float32


def _gelu(x):
    return 0.5 * x * (1.0 + jax.lax.erf(x * (2.0 ** -0.5)))


def _ln(x, g=None, b=None):
    m = jnp.mean(x, axis=-1, keepdims=True)
    xc = x - m
    v = jnp.mean(xc * xc, axis=-1, keepdims=True)
    y = xc * jax.lax.rsqrt(v + _EPS)
    if g is not None:
        y = y * g + b
    return y


def _mm_t(a, w):
    return jax.lax.dot_general(a, w, (((1,), (1,)), ((), ())),
                               preferred_element_type=_F32)


def _fused_kernel(x_ref,
                  sw1, sb1, sw2, sb2, sw3, sb3, sg, sbe,
                  ow1, ob1, ow2, ob2, ow3, ob3, og, obe,
                  w1, b1, w2, b2, g, be, cw, cb, bw, bb,
                  probs_ref, bbox_ref):
    x = x_ref[0]

    def head(hw1, hb1, hw2, hb2, hw3, hb3, hg, hbe):
        h = _gelu(_mm_t(x, hw1[...]) + hb1[...])
        h = _gelu(_mm_t(h, hw2[...]) + hb2[...])
        h = _gelu(_mm_t(h, hw3[...]) + hb3[...])
        return _ln(h, hg[...], hbe[...])

    q = x + head(sw1, sb1, sw2, sb2, sw3, sb3, sg, sbe)
    k = x + head(ow1, ob1, ow2, ob2, ow3, ob3, og, obe)

    st = jax.lax.dot_general(k, q, (((1,), (1,)), ((), ())),
                             preferred_element_type=_F32)
    ii = jax.lax.broadcasted_iota(jnp.int32, (_N, _N), 0)
    jj = jax.lax.broadcasted_iota(jnp.int32, (_N, _N), 1)
    m = jnp.max(st, axis=0, keepdims=True)
    z = jnp.sum(jnp.exp(st - m), axis=0, keepdims=True)
    sd = jnp.sum(jnp.where(ii == jj, st, 0.0), axis=0, keepdims=True)
    d = jnp.exp(sd - m) / z

    diag_d = jnp.where(ii == jj, d, 0.0)
    dcol = jnp.dot(diag_d, jnp.ones((_N, 128), _F32),
                   preferred_element_type=_F32)
    di = pltpu.repeat(dcol, 8, axis=1)
    beats1 = jnp.where((d > di) | ((d == di) & (jj < ii)), 1.0, 0.0)
    rank_col = jnp.dot(beats1, jnp.ones((_N, 128), _F32),
                       preferred_element_type=_F32)
    sel_col = jnp.where(rank_col < float(_K), 1.0, 0.0)
    sel_coln = pltpu.repeat(sel_col, 8, axis=1)
    beats2 = jnp.where(ii == jj, 0.0, 1.0 - beats1)
    rank_row = jnp.sum(beats2, axis=0, keepdims=True)
    sel_row = jnp.where(rank_row < float(_K), 1.0, 0.0)
    psel = jnp.sum(jnp.where(ii <= jj, sel_coln, 0.0),
                   axis=0, keepdims=True)

    ss = jax.lax.broadcasted_iota(jnp.int32, (_K, _N), 0).astype(_F32)
    oh = jnp.where((sel_row > 0.0) & (psel == ss + 1.0), 1.0, 0.0)
    q_sel = jnp.dot(oh, q, preferred_element_type=_F32)

    h = _ln(q_sel + q_sel)
    h = _gelu(_mm_t(h, w1[...]) + b1[...])
    h = _mm_t(h, w2[...]) + b2[...]
    o = _ln(h, g[...], be[...])

    bbox_ref[0] = jax.nn.relu(_mm_t(o, bw[...]) + bb[...])
    lg = _mm_t(o, cw[...]) + cb[...]
    lm = jnp.max(lg, axis=-1, keepdims=True)
    e = jnp.exp(lg - lm)
    probs_ref[0] = e / jnp.sum(e, axis=-1, keepdims=True)


def _full_spec(shape):
    return pl.BlockSpec(shape, lambda *_: (0,) * len(shape))


def kernel(x, params):
    sh, oh_, m2 = params['subject_head'], params['object_head'], params['mlp2']
    r = lambda v: v.reshape(1, -1)

    w_args = []
    specs_w = []
    for p in (sh, oh_):
        for i in (1, 2, 3):
            w_args += [p[f'w{i}'], r(p[f'b{i}'])]
            specs_w += [_full_spec((_D, _D)), _full_spec((1, _D))]
        w_args += [r(p['g']), r(p['be'])]
        specs_w += [_full_spec((1, _D)), _full_spec((1, _D))]

    probs, bbox = pl.pallas_call(
        _fused_kernel,
        grid=(_B,),
        in_specs=[pl.BlockSpec((1, _N, _D), lambda b: (b, 0, 0))] + specs_w + [
            _full_spec((_D, _D)), _full_spec((1, _D)),
            _full_spec((_D, _D)), _full_spec((1, _D)),
            _full_spec((1, _D)), _full_spec((1, _D)),
            _full_spec((_C, _D)), _full_spec((1, _C)),
            _full_spec((4, _D)), _full_spec((1, 4)),
        ],
        out_specs=[pl.BlockSpec((1, _K, _C), lambda b: (b, 0, 0)),
                   pl.BlockSpec((1, _K, 4), lambda b: (b, 0, 0))],
        out_shape=[jax.ShapeDtypeStruct((_B, _K, _C), _F32),
                   jax.ShapeDtypeStruct((_B, _K, 4), _F32)],
        compiler_params=pltpu.CompilerParams(
            dimension_semantics=("parallel",),
            vmem_limit_bytes=58 * 1024 * 1024,
        ),
    )(x, *w_args, m2['w1'], r(m2['b1']), m2['w2'], r(m2['b2']),
      r(m2['g']), r(m2['be']),
      params['cls_w'], r(params['cls_b']), params['bbox_w'], r(params['bbox_b']))

    return probs, bbox
